# Initial kernel scaffold; baseline (speedup 1.0000x reference)
import jax
import jax.numpy as jnp
from jax.experimental import pallas as pl


def kernel(labels, features):
    raise NotImplementedError("write your pallas kernel here")


# speedup vs baseline: 470.9897x; 470.9897x over previous
import numpy as np
import jax
import jax.numpy as jnp
from jax.experimental import pallas as pl
from jax.experimental.pallas import tpu as pltpu

NEG = -1e30
INF = 1 << 30
THRESH = np.arange(1, 0, -0.05).astype(np.float32)
NT = THRESH.shape[0]


def _tc_body(plab_ref, labs_ref, poss_ref, f_ref, fs_ref, out_ref, m_acc, accum,
             *, rows, n, nkc, kc_size, g_tiles):
    g = pl.program_id(0)
    kc = pl.program_id(1)

    prod = jax.lax.dot_general(
        f_ref[...], fs_ref[...],
        (((1,), (1,)), ((), ())),
        preferred_element_type=jnp.float32,
        precision=jax.lax.Precision.HIGHEST)
    m_acc[:, pl.ds(kc * kc_size, kc_size)] = prod

    @pl.when(kc == nkc - 1)
    def _():
        m = m_acc[...]
        plab = plab_ref[0:1, :]
        labrow = labs_ref[0]
        posrow = poss_ref[0]

        ri = jax.lax.broadcasted_iota(jnp.int32, (rows, rows), 0)
        ci = jax.lax.broadcasted_iota(jnp.int32, (rows, rows), 1)
        eye = ri == ci
        labcol = jnp.sum(jnp.where(eye, jnp.broadcast_to(labrow, (rows, rows)), 0),
                         axis=1, keepdims=True)
        poscol = jnp.sum(jnp.where(eye, jnp.broadcast_to(posrow, (rows, rows)), 0),
                         axis=1, keepdims=True)

        lane = jax.lax.broadcasted_iota(jnp.int32, (1, n), 1)
        m = jnp.where(lane == poscol, -1000.0, m)

        sL = jnp.max(jnp.where(plab == labcol, m, NEG), axis=1, keepdims=True)
        s0 = jnp.max(jnp.where(plab == 0, m, NEG), axis=1, keepdims=True)
        zero_present = plab_ref[0, 0] == 0

        v = m
        s = 1
        while s < n:
            vsh = jnp.concatenate(
                [jnp.full((rows, s), NEG, jnp.float32), v[:, :n - s]], axis=1)
            psh = jnp.concatenate(
                [jnp.full((1, s), -1, jnp.int32), plab[:, :n - s]], axis=1)
            v = jnp.maximum(v, jnp.where(psh == plab, vsh, NEG))
            s *= 2

        pnext = jnp.concatenate([plab[:, 1:], jnp.full((1, 1), -1, jnp.int32)], axis=1)
        is_end = plab != pnext
        big = jnp.where(is_end, v, NEG)

        q = jnp.sum((big > sL).astype(jnp.int32), axis=1, keepdims=True)
        zc = jnp.sum((big > s0).astype(jnp.int32), axis=1, keepdims=True)
        z = jnp.where(zero_present, zc, INF)

        lane128 = jax.lax.broadcasted_iota(jnp.int32, (1, 128), 1)
        accvec = jnp.zeros((1, 128), jnp.float32)
        for t_idx in range(NT):
            t = THRESH[t_idx]
            c_t = jnp.sum((big >= t).astype(jnp.int32), axis=1, keepdims=True)
            extra = ((z >= c_t) & (q >= c_t) & (q < z)).astype(jnp.int32)
            rank = q + extra
            apk = jnp.where((labcol == 0) | (rank >= 5),
                            0.0, 1.0 / (rank.astype(jnp.float32) + 1.0))
            accvec += jnp.where(lane128 == t_idx, jnp.sum(apk), 0.0)

        accum[...] = jnp.where(g == 0, accvec, accum[...] + accvec)

        @pl.when(g == g_tiles - 1)
        def _():
            acc = accum[...]
            masked = jnp.where(lane128 < NT, acc / n, NEG)
            out_ref[...] = jnp.full((8, 128), jnp.max(masked), jnp.float32)


def _map_pallas(plab, labs3, poss3, f, fs):
    n, d = f.shape
    rows = 128 if n % 128 == 0 else n
    g_tiles = n // rows
    kc_size = 512 if n % 512 == 0 else n
    nkc = n // kc_size

    import functools
    body = functools.partial(_tc_body, rows=rows, n=n, nkc=nkc,
                             kc_size=kc_size, g_tiles=g_tiles)
    return pl.pallas_call(
        body,
        grid=(g_tiles, nkc),
        in_specs=[
            pl.BlockSpec((1, n), lambda g, kc: (0, 0)),
            pl.BlockSpec((1, 1, rows), lambda g, kc: (g, 0, 0)),
            pl.BlockSpec((1, 1, rows), lambda g, kc: (g, 0, 0)),
            pl.BlockSpec((rows, d), lambda g, kc: (g, 0)),
            pl.BlockSpec((kc_size, d), lambda g, kc: (kc, 0)),
        ],
        out_specs=pl.BlockSpec((8, 128), lambda g, kc: (0, 0)),
        out_shape=jax.ShapeDtypeStruct((8, 128), jnp.float32),
        scratch_shapes=[
            pltpu.VMEM((rows, n), jnp.float32),
            pltpu.VMEM((1, 128), jnp.float32),
        ],
        compiler_params=pltpu.CompilerParams(
            dimension_semantics=("arbitrary", "arbitrary")),
    )(plab, labs3, poss3, f, fs)


def kernel(labels, features):
    n, d = features.shape
    rows = 128 if n % 128 == 0 else n
    g_tiles = n // rows

    perm = jnp.argsort(labels)
    plab = labels[perm].reshape(1, n)
    fs = features[perm]
    pos = jnp.zeros((n,), jnp.int32).at[perm].set(jnp.arange(n, dtype=jnp.int32))

    labs3 = labels.reshape(g_tiles, 1, rows)
    poss3 = pos.reshape(g_tiles, 1, rows)

    out = _map_pallas(plab, labs3, poss3, features, fs)
    return out[0, 0]


# speedup vs baseline: 659.7891x; 1.4009x over previous
import functools
import numpy as np
import jax
import jax.numpy as jnp
from jax.experimental import pallas as pl
from jax.experimental.pallas import tpu as pltpu

NEG = -1e30
THRESH = np.arange(1, 0, -0.05).astype(np.float32)
NT = THRESH.shape[0]
CPAD = 1024


def _tc_body(plab_ref, plabT_ref, labs_ref, poss_ref, f_ref, fs_ref, out_ref,
             m_acc, onehot, accum, *, rows, n, nkc, kc_size, g_tiles):
    g = pl.program_id(0)
    kc = pl.program_id(1)

    @pl.when(g == 0)
    def _():
        pcol = plabT_ref[0]
        lanec = jax.lax.broadcasted_iota(jnp.int32, (1, CPAD), 1)
        onehot[pl.ds(kc * kc_size, kc_size), :] = (pcol == lanec).astype(jnp.bfloat16)

    prod = jax.lax.dot_general(
        f_ref[...], fs_ref[...],
        (((1,), (1,)), ((), ())),
        preferred_element_type=jnp.float32,
        precision=jax.lax.Precision.HIGHEST)
    m_acc[:, pl.ds(kc * kc_size, kc_size)] = prod

    @pl.when(kc == nkc - 1)
    def _():
        m = m_acc[...]
        plab = plab_ref[0:1, :]
        labrow = labs_ref[0]
        posrow = poss_ref[0]

        ri = jax.lax.broadcasted_iota(jnp.int32, (rows, rows), 0)
        ci = jax.lax.broadcasted_iota(jnp.int32, (rows, rows), 1)
        eye = ri == ci
        labcol = jnp.sum(jnp.where(eye, jnp.broadcast_to(labrow, (rows, rows)), 0),
                         axis=1, keepdims=True)
        poscol = jnp.sum(jnp.where(eye, jnp.broadcast_to(posrow, (rows, rows)), 0),
                         axis=1, keepdims=True)

        lane = jax.lax.broadcasted_iota(jnp.int32, (1, n), 1)
        m = jnp.where(lane == poscol, -1000.0, m)

        sL = jnp.max(jnp.where(plab == labcol, m, NEG), axis=1, keepdims=True)
        s0 = jnp.max(jnp.where(plab == 0, m, NEG), axis=1, keepdims=True)

        ind = (m > sL).astype(jnp.bfloat16)
        cnt = jax.lax.dot_general(
            ind, onehot[...],
            (((1,), (0,)), ((), ())),
            preferred_element_type=jnp.float32)
        q = jnp.sum((cnt >= 0.5).astype(jnp.int32), axis=1, keepdims=True)

        s0lt = s0 < sL
        lane128 = jax.lax.broadcasted_iota(jnp.int32, (1, 128), 1)
        accvec = jnp.zeros((1, 128), jnp.float32)
        for t_idx in range(NT):
            t = THRESH[t_idx]
            extra = ((sL < t) & s0lt).astype(jnp.int32)
            rank = q + extra
            apk = jnp.where((labcol == 0) | (rank >= 5),
                            0.0, 1.0 / (rank.astype(jnp.float32) + 1.0))
            accvec += jnp.where(lane128 == t_idx, jnp.sum(apk), 0.0)

        accum[...] = jnp.where(g == 0, accvec, accum[...] + accvec)

        @pl.when(g == g_tiles - 1)
        def _():
            masked = jnp.where(lane128 < NT, accum[...] / n, NEG)
            out_ref[...] = jnp.full((8, 128), jnp.max(masked), jnp.float32)


def _map_pallas(plab, plabT3, labs3, poss3, f, fs):
    n, d = f.shape
    rows = 128 if n % 128 == 0 else n
    g_tiles = n // rows
    kc_size = 512 if n % 512 == 0 else n
    nkc = n // kc_size

    body = functools.partial(_tc_body, rows=rows, n=n, nkc=nkc,
                             kc_size=kc_size, g_tiles=g_tiles)
    return pl.pallas_call(
        body,
        grid=(g_tiles, nkc),
        in_specs=[
            pl.BlockSpec((1, n), lambda g, kc: (0, 0)),
            pl.BlockSpec((1, kc_size, 1), lambda g, kc: (kc, 0, 0)),
            pl.BlockSpec((1, 1, rows), lambda g, kc: (g, 0, 0)),
            pl.BlockSpec((1, 1, rows), lambda g, kc: (g, 0, 0)),
            pl.BlockSpec((rows, d), lambda g, kc: (g, 0)),
            pl.BlockSpec((kc_size, d), lambda g, kc: (kc, 0)),
        ],
        out_specs=pl.BlockSpec((8, 128), lambda g, kc: (0, 0)),
        out_shape=jax.ShapeDtypeStruct((8, 128), jnp.float32),
        scratch_shapes=[
            pltpu.VMEM((rows, n), jnp.float32),
            pltpu.VMEM((n, CPAD), jnp.bfloat16),
            pltpu.VMEM((1, 128), jnp.float32),
        ],
        compiler_params=pltpu.CompilerParams(
            dimension_semantics=("arbitrary", "arbitrary")),
    )(plab, plabT3, labs3, poss3, f, fs)


def kernel(labels, features):
    n, d = features.shape
    rows = 128 if n % 128 == 0 else n
    g_tiles = n // rows
    kc_size = 512 if n % 512 == 0 else n
    nkc = n // kc_size

    perm = jnp.argsort(labels)
    plab_s = labels[perm]
    plab = plab_s.reshape(1, n)
    plabT3 = plab_s.reshape(nkc, kc_size, 1)
    fs = features[perm]
    pos = jnp.zeros((n,), jnp.int32).at[perm].set(jnp.arange(n, dtype=jnp.int32))

    labs3 = labels.reshape(g_tiles, 1, rows)
    poss3 = pos.reshape(g_tiles, 1, rows)

    out = _map_pallas(plab, plabT3, labs3, poss3, features, fs)
    return out[0, 0]


# speedup vs baseline: 2339.0003x; 3.5451x over previous
import functools
import numpy as np
import jax
import jax.numpy as jnp
from jax.experimental import pallas as pl
from jax.experimental.pallas import tpu as pltpu

NEG = -1e30
THRESH = np.arange(1, 0, -0.05).astype(np.float32)
NT = THRESH.shape[0]
CPAD = 1024


def _tc_body(labrow_ref, labT_ref, f_ref, fr_ref, out_ref, onehot, accum,
             *, rows, n, g_tiles):
    g = pl.program_id(0)

    @pl.when(g == 0)
    def _():
        lanec = jax.lax.broadcasted_iota(jnp.int32, (1, CPAD), 1)
        onehot[...] = (labT_ref[...] == lanec).astype(jnp.bfloat16)

    m = jax.lax.dot_general(
        f_ref[...], fr_ref[...],
        (((1,), (1,)), ((), ())),
        preferred_element_type=jnp.float32,
        precision=jax.lax.Precision.DEFAULT)

    labrow = labrow_ref[0:1, :]
    labcol = labT_ref[pl.ds(g * rows, rows), :]

    lane = jax.lax.broadcasted_iota(jnp.int32, (1, n), 1)
    rowid = jax.lax.broadcasted_iota(jnp.int32, (rows, 1), 0) + g * rows
    m = jnp.where(lane == rowid, -1000.0, m)

    labmask = labrow == labcol
    zmask = labrow == 0
    sL = jnp.max(jnp.where(labmask, m, NEG), axis=1, keepdims=True)
    s0 = jnp.max(jnp.where(zmask, m, NEG), axis=1, keepdims=True)

    jmaxL = jnp.max(jnp.where(labmask & (m == sL), lane, -1),
                    axis=1, keepdims=True)
    jmax0 = jnp.max(jnp.where(zmask & (m == s0), lane, -1),
                    axis=1, keepdims=True)

    ind = ((m > sL) | ((m == sL) & (lane > jmaxL))).astype(jnp.bfloat16)
    cnt = jax.lax.dot_general(
        ind, onehot[...],
        (((1,), (0,)), ((), ())),
        preferred_element_type=jnp.float32)
    q = jnp.sum((cnt >= 0.5).astype(jnp.int32), axis=1, keepdims=True)

    s0lt = (s0 < sL) | ((s0 == sL) & (jmax0 < jmaxL))

    lane128 = jax.lax.broadcasted_iota(jnp.int32, (1, 128), 1)
    accvec = jnp.zeros((1, 128), jnp.float32)
    for t_idx in range(NT):
        t = THRESH[t_idx]
        extra = ((sL < t) & s0lt).astype(jnp.int32)
        rank = q + extra
        apk = jnp.where((labcol == 0) | (rank >= 5),
                        0.0, 1.0 / (rank.astype(jnp.float32) + 1.0))
        accvec += jnp.where(lane128 == t_idx, jnp.sum(apk), 0.0)

    accum[...] = jnp.where(g == 0, accvec, accum[...] + accvec)

    @pl.when(g == g_tiles - 1)
    def _():
        masked = jnp.where(lane128 < NT, accum[...] / n, NEG)
        out_ref[...] = jnp.full((8, 128), jnp.max(masked), jnp.float32)


def kernel(labels, features):
    n, d = features.shape
    rows = 128 if n % 128 == 0 else n
    g_tiles = n // rows

    labrow = labels.reshape(1, n)
    labT = labels.reshape(n, 1)

    body = functools.partial(_tc_body, rows=rows, n=n, g_tiles=g_tiles)
    out = pl.pallas_call(
        body,
        grid=(g_tiles,),
        in_specs=[
            pl.BlockSpec((1, n), lambda g: (0, 0)),
            pl.BlockSpec((n, 1), lambda g: (0, 0)),
            pl.BlockSpec((rows, d), lambda g: (g, 0)),
            pl.BlockSpec((n, d), lambda g: (0, 0)),
        ],
        out_specs=pl.BlockSpec((8, 128), lambda g: (0, 0)),
        out_shape=jax.ShapeDtypeStruct((8, 128), jnp.float32),
        scratch_shapes=[
            pltpu.VMEM((n, CPAD), jnp.bfloat16),
            pltpu.VMEM((1, 128), jnp.float32),
        ],
        compiler_params=pltpu.CompilerParams(
            dimension_semantics=("arbitrary",)),
    )(labrow, labT, features, features)
    return out[0, 0]


# speedup vs baseline: 3126.9586x; 1.3369x over previous
import functools
import numpy as np
import jax
import jax.numpy as jnp
from jax.experimental import pallas as pl
from jax.experimental.pallas import tpu as pltpu

NEG = -1e30
THRESH = np.arange(1, 0, -0.05).astype(np.float32)
NT = THRESH.shape[0]
CPAD = 1024


def _tc_body(labrow_ref, labT_ref, f_ref, fr_ref, out_ref, onehot, accum,
             *, rows, n, g_tiles):
    g = pl.program_id(0)

    @pl.when(g == 0)
    def _():
        lanec = jax.lax.broadcasted_iota(jnp.int32, (1, CPAD), 1)
        onehot[...] = (labT_ref[...] == lanec).astype(jnp.bfloat16)

    m = jax.lax.dot_general(
        f_ref[...], fr_ref[...],
        (((1,), (1,)), ((), ())),
        preferred_element_type=jnp.float32,
        precision=jax.lax.Precision.DEFAULT)

    labrow = labrow_ref[0:1, :]
    labcol = labT_ref[pl.ds(g * rows, rows), :]

    lane = jax.lax.broadcasted_iota(jnp.int32, (1, n), 1)
    rowid = jax.lax.broadcasted_iota(jnp.int32, (rows, 1), 0) + g * rows
    m = jnp.where(lane == rowid, -1000.0, m)

    labmask = labrow == labcol
    zmask = labrow == 0
    sL = jnp.max(jnp.where(labmask, m, NEG), axis=1, keepdims=True)
    s0 = jnp.max(jnp.where(zmask, m, NEG), axis=1, keepdims=True)

    jmaxL = jnp.max(jnp.where(labmask & (m == sL), lane, -1),
                    axis=1, keepdims=True)
    jmax0 = jnp.max(jnp.where(zmask & (m == s0), lane, -1),
                    axis=1, keepdims=True)

    ind = ((m > sL) | ((m == sL) & (lane > jmaxL))).astype(jnp.bfloat16)
    cnt = jax.lax.dot_general(
        ind, onehot[...],
        (((1,), (0,)), ((), ())),
        preferred_element_type=jnp.float32)
    q = jnp.sum((cnt >= 0.5).astype(jnp.int32), axis=1, keepdims=True)

    s0lt = (s0 < sL) | ((s0 == sL) & (jmax0 < jmaxL))

    lane128 = jax.lax.broadcasted_iota(jnp.int32, (1, 128), 1)
    accvec = jnp.zeros((1, 128), jnp.float32)
    for t_idx in range(NT):
        t = THRESH[t_idx]
        extra = ((sL < t) & s0lt).astype(jnp.int32)
        rank = q + extra
        apk = jnp.where((labcol == 0) | (rank >= 5),
                        0.0, 1.0 / (rank.astype(jnp.float32) + 1.0))
        accvec += jnp.where(lane128 == t_idx, jnp.sum(apk), 0.0)

    accum[...] = jnp.where(g == 0, accvec, accum[...] + accvec)

    @pl.when(g == g_tiles - 1)
    def _():
        masked = jnp.where(lane128 < NT, accum[...] / n, NEG)
        out_ref[...] = jnp.full((8, 128), jnp.max(masked), jnp.float32)


def kernel(labels, features):
    n, d = features.shape
    rows = 256 if n % 256 == 0 else n
    g_tiles = n // rows

    labrow = labels.reshape(1, n)
    labT = labels.reshape(n, 1)
    fb = features.astype(jnp.bfloat16)

    body = functools.partial(_tc_body, rows=rows, n=n, g_tiles=g_tiles)
    out = pl.pallas_call(
        body,
        grid=(g_tiles,),
        in_specs=[
            pl.BlockSpec((1, n), lambda g: (0, 0)),
            pl.BlockSpec((n, 1), lambda g: (0, 0)),
            pl.BlockSpec((rows, d), lambda g: (g, 0)),
            pl.BlockSpec((n, d), lambda g: (0, 0)),
        ],
        out_specs=pl.BlockSpec((8, 128), lambda g: (0, 0)),
        out_shape=jax.ShapeDtypeStruct((8, 128), jnp.float32),
        scratch_shapes=[
            pltpu.VMEM((n, CPAD), jnp.bfloat16),
            pltpu.VMEM((1, 128), jnp.float32),
        ],
        compiler_params=pltpu.CompilerParams(
            dimension_semantics=("arbitrary",)),
    )(labrow, labT, fb, fb)
    return out[0, 0]


# speedup vs baseline: 3408.2104x; 1.0899x over previous
import functools
import numpy as np
import jax
import jax.numpy as jnp
from jax.experimental import pallas as pl
from jax.experimental.pallas import tpu as pltpu

NEG = -1e30
THRESH = np.arange(1, 0, -0.05).astype(np.float32)
NT = THRESH.shape[0]
CPAD = 1024


def _tc_body(labrow_ref, labT_ref, f_ref, fr_ref, out_ref, onehot, accum,
             *, rows, n, g_tiles):
    g = pl.program_id(0)

    @pl.when(g == 0)
    def _():
        lanec = jax.lax.broadcasted_iota(jnp.int32, (1, CPAD), 1)
        onehot[...] = (labT_ref[...] == lanec).astype(jnp.bfloat16)

    m = jax.lax.dot_general(
        f_ref[...], fr_ref[...],
        (((1,), (1,)), ((), ())),
        preferred_element_type=jnp.float32,
        precision=jax.lax.Precision.DEFAULT)

    labrow = labrow_ref[0:1, :]
    labcol = labT_ref[pl.ds(g * rows, rows), :]

    lane = jax.lax.broadcasted_iota(jnp.int32, (1, n), 1)
    rowid = jax.lax.broadcasted_iota(jnp.int32, (rows, 1), 0) + g * rows
    m = jnp.where(lane == rowid, -1000.0, m)

    labmask = labrow == labcol
    zmask = labrow == 0
    sL = jnp.max(jnp.where(labmask, m, NEG), axis=1, keepdims=True)
    s0 = jnp.max(jnp.where(zmask, m, NEG), axis=1, keepdims=True)

    jmaxL = jnp.max(jnp.where(labmask & (m == sL), lane, -1),
                    axis=1, keepdims=True)
    jmax0 = jnp.max(jnp.where(zmask & (m == s0), lane, -1),
                    axis=1, keepdims=True)

    ind = ((m > sL) | ((m == sL) & (lane > jmaxL))).astype(jnp.bfloat16)
    cnt = jax.lax.dot_general(
        ind, onehot[...],
        (((1,), (0,)), ((), ())),
        preferred_element_type=jnp.float32)
    q = jnp.sum((cnt >= 0.5).astype(jnp.int32), axis=1, keepdims=True)

    s0lt = (s0 < sL) | ((s0 == sL) & (jmax0 < jmaxL))

    lane128 = jax.lax.broadcasted_iota(jnp.int32, (1, 128), 1)
    accvec = jnp.zeros((1, 128), jnp.float32)
    for t_idx in range(NT):
        t = THRESH[t_idx]
        extra = ((sL < t) & s0lt).astype(jnp.int32)
        rank = q + extra
        apk = jnp.where((labcol == 0) | (rank >= 5),
                        0.0, 1.0 / (rank.astype(jnp.float32) + 1.0))
        accvec += jnp.where(lane128 == t_idx, jnp.sum(apk), 0.0)

    accum[...] = jnp.where(g == 0, accvec, accum[...] + accvec)

    @pl.when(g == g_tiles - 1)
    def _():
        masked = jnp.where(lane128 < NT, accum[...] / n, NEG)
        out_ref[...] = jnp.full((8, 128), jnp.max(masked), jnp.float32)


def kernel(labels, features):
    n, d = features.shape
    rows = 1024 if n % 1024 == 0 else n
    g_tiles = n // rows

    labrow = labels.reshape(1, n)
    labT = labels.reshape(n, 1)
    fb = features.astype(jnp.bfloat16)

    body = functools.partial(_tc_body, rows=rows, n=n, g_tiles=g_tiles)
    out = pl.pallas_call(
        body,
        grid=(g_tiles,),
        in_specs=[
            pl.BlockSpec((1, n), lambda g: (0, 0)),
            pl.BlockSpec((n, 1), lambda g: (0, 0)),
            pl.BlockSpec((rows, d), lambda g: (g, 0)),
            pl.BlockSpec((n, d), lambda g: (0, 0)),
        ],
        out_specs=pl.BlockSpec((8, 128), lambda g: (0, 0)),
        out_shape=jax.ShapeDtypeStruct((8, 128), jnp.float32),
        scratch_shapes=[
            pltpu.VMEM((n, CPAD), jnp.bfloat16),
            pltpu.VMEM((1, 128), jnp.float32),
        ],
        compiler_params=pltpu.CompilerParams(
            dimension_semantics=("arbitrary",)),
    )(labrow, labT, fb, fb)
    return out[0, 0]
